# baseline (speedup 1.0000x reference)
import jax
import jax.numpy as jnp
from jax import lax
from jax.experimental import pallas as pl
from jax.experimental.pallas import tpu as pltpu
from jax.experimental.pallas import tpu_sc as plsc

N = 10000
H = 128
C = 7
NNZ = 320000

NSUB = 16
EPT = NNZ // NSUB
ECH = 80
NCH = EPT // ECH
RPT = 624
TAIL = N - NSUB * RPT

BN = 2000
GRID = N // BN

f32 = jnp.float32
i32 = jnp.int32


def _sc_edge_body(rows_u, cols_u, vals_u, rows_d, cols_d, vals_d, h2, abp,
                  z_out, den_out,
                  ab_t, rows_b, cols_b, vals_b, srows_b, pbuf, coef_b,
                  gath_b, zvec, zacc, den_sh,
                  psem0, psem1, gsem0, gsem1, gsem2, ssem0, ssem1, ssem2):
    core = lax.axis_index("c")
    s = lax.axis_index("s")
    psem = (psem0, psem1)
    gsem = (gsem0, gsem1, gsem2)
    ssem = (ssem0, ssem1, ssem2)

    def _zv(i, _):
        zvec[pl.ds(i * 16, 16)] = jnp.zeros((16,), f32)
        return 0
    lax.fori_loop(0, RPT // 16, _zv, 0)

    def _zg(i, _):
        r = i // 8
        j = i - r * 8
        gath_b[0, r, pl.ds(j * 16, 16)] = jnp.zeros((16,), f32)
        return 0
    lax.fori_loop(0, 64, _zg, 0)

    pltpu.sync_copy(zvec, den_sh.at[pl.ds(s * RPT, RPT)])

    def _zc(k, _):
        pltpu.sync_copy(gath_b.at[0, pl.ds(0, 8)],
                        zacc.at[pl.ds(s * RPT + k * 8, 8)])
        return 0
    lax.fori_loop(0, RPT // 8, _zc, 0)

    @pl.when(s == NSUB - 1)
    def _():
        pltpu.sync_copy(zvec.at[pl.ds(0, TAIL)],
                        den_sh.at[pl.ds(NSUB * RPT, TAIL)])
        for t in range(TAIL // 8):
            pltpu.sync_copy(gath_b.at[0, pl.ds(0, 8)],
                            zacc.at[pl.ds(NSUB * RPT + t * 8, 8)])

    plsc.subcore_barrier()

    def run_dir(d, rows_h, cols_h, vals_h):
        pltpu.sync_copy(abp.at[pl.ds(d * N, N)], ab_t)

        def pack_start(k, b):
            base = s * EPT + k * ECH
            pltpu.async_copy(rows_h.at[pl.ds(base, ECH)], rows_b.at[b],
                             psem[b])
            pltpu.async_copy(cols_h.at[pl.ds(base, ECH)], cols_b.at[b],
                             psem[b])
            pltpu.async_copy(vals_h.at[pl.ds(base, ECH)], vals_b.at[b],
                             psem[b])

        def pack_wait(b):
            base = s * EPT
            pltpu.make_async_copy(rows_h.at[pl.ds(base, ECH)],
                                  rows_b.at[b], psem[b]).wait()
            pltpu.make_async_copy(cols_h.at[pl.ds(base, ECH)],
                                  cols_b.at[b], psem[b]).wait()
            pltpu.make_async_copy(vals_h.at[pl.ds(base, ECH)],
                                  vals_b.at[b], psem[b]).wait()

        def gather_start(g, b):
            pltpu.async_copy(h2.at[d].at[cols_b.at[b]], gath_b.at[g],
                             gsem[g])

        def gather_wait(g, b):
            pltpu.make_async_copy(h2.at[d].at[cols_b.at[b]], gath_b.at[g],
                                  gsem[g]).wait()

        def scatter_start(g, b):
            pltpu.async_copy(gath_b.at[g], zacc.at[srows_b.at[b]], ssem[g],
                             add=True)
            pltpu.async_copy(pbuf.at[b], den_sh.at[srows_b.at[b]], ssem[g],
                             add=True)

        def scatter_wait(g, b):
            pltpu.make_async_copy(gath_b.at[g], zacc.at[srows_b.at[b]],
                                  ssem[g]).wait()
            pltpu.make_async_copy(pbuf.at[b], den_sh.at[srows_b.at[b]],
                                  ssem[g]).wait()

        def dispatch3(gv, fn):
            for v in range(3):
                @pl.when(gv == v)
                def _():
                    fn(v)

        def compute(b):
            for g in range(ECH // 16):
                sl = pl.ds(g * 16, 16)
                r16 = rows_b[b, sl]
                c16 = cols_b[b, sl]
                srows_b[b, sl] = r16
                tr = plsc.load_gather(ab_t, [r16])
                tc_ = plsc.load_gather(ab_t, [c16])
                av = plsc.bitcast(lax.shift_left(tr, 16), f32)
                bv = plsc.bitcast(
                    jnp.bitwise_and(tc_, jnp.int32(-65536)), f32)
                e = av + bv
                e = jnp.maximum(e, 0.2 * e)
                p = jnp.exp(e)
                pbuf[b, sl] = p
                coef_b[sl] = p * vals_b[b, sl]

        def scale(gv):
            def body(e_i, _):
                g = e_i // 16
                lane = e_i - g * 16
                cvec = coef_b[pl.ds(g * 16, 16)]
                bc = lax.gather(
                    cvec, jnp.full((16, 1), lane, i32),
                    dimension_numbers=lax.GatherDimensionNumbers(
                        offset_dims=(), collapsed_slice_dims=(0,),
                        start_index_map=(0,)),
                    slice_sizes=(1,),
                    mode=lax.GatherScatterMode.PROMISE_IN_BOUNDS)
                for j in range(8):
                    sl2 = pl.ds(j * 16, 16)
                    gath_b[gv, e_i, sl2] = gath_b[gv, e_i, sl2] * bc
                return 0
            lax.fori_loop(0, ECH, body, 0, unroll=4)

        pack_start(0, 0)
        pack_start(1, 1)
        pack_wait(0)
        gather_start(0, 0)

        def loop_body(kk, _):
            for j in range(2):
                b, opp = j, 1 - j
                k = 2 * kk + j
                gv = k % 3

                @pl.when(kk > 0)
                def _():
                    dispatch3((k - 2) % 3,
                              lambda v: scatter_wait(v, b))
                compute(b)
                dispatch3(gv, lambda v: gather_wait(v, b))

                @pl.when(kk < NCH // 2 - 1)
                def _():
                    pack_start(k + 2, b)

                if j == 0:
                    pack_wait(opp)
                    dispatch3((k + 1) % 3,
                              lambda v: gather_start(v, opp))
                else:
                    @pl.when(kk < NCH // 2 - 1)
                    def _():
                        pack_wait(opp)
                        dispatch3((k + 1) % 3,
                                  lambda v: gather_start(v, opp))
                dispatch3(gv, scale)
                dispatch3(gv, lambda v: scatter_start(v, b))
            return 0

        lax.fori_loop(0, NCH // 2, loop_body, 0)
        scatter_wait((NCH - 2) % 3, 0)
        scatter_wait((NCH - 1) % 3, 1)

        plsc.subcore_barrier()
        pltpu.sync_copy(zacc.at[pl.ds(s * RPT, RPT)],
                        z_out.at[d, pl.ds(s * RPT, RPT)])
        pltpu.sync_copy(den_sh.at[pl.ds(s * RPT, RPT)], zvec)
        pltpu.sync_copy(zvec, den_out.at[pl.ds(d * N + s * RPT, RPT)])

        @pl.when(s == NSUB - 1)
        def _():
            pltpu.sync_copy(zacc.at[pl.ds(NSUB * RPT, TAIL)],
                            z_out.at[d, pl.ds(NSUB * RPT, TAIL)])
            pltpu.sync_copy(den_sh.at[pl.ds(NSUB * RPT, TAIL)],
                            zvec.at[pl.ds(0, TAIL)])
            pltpu.sync_copy(zvec.at[pl.ds(0, TAIL)],
                            den_out.at[pl.ds(d * N + NSUB * RPT, TAIL)])

    @pl.when(core == 0)
    def _():
        run_dir(0, rows_u, cols_u, vals_u)

    @pl.when(core == 1)
    def _():
        run_dir(1, rows_d, cols_d, vals_d)


_sc_edge = pl.kernel(
    _sc_edge_body,
    out_type=(
        jax.ShapeDtypeStruct((2, N, H), f32),
        jax.ShapeDtypeStruct((2 * N,), f32),
    ),
    mesh=plsc.VectorSubcoreMesh(core_axis_name="c", subcore_axis_name="s"),
    compiler_params=pltpu.CompilerParams(needs_layout_passes=False),
    scratch_types=[
        pltpu.VMEM((N,), i32),
        pltpu.VMEM((2, ECH), i32),
        pltpu.VMEM((2, ECH), i32),
        pltpu.VMEM((2, ECH), f32),
        pltpu.VMEM((2, ECH), i32),
        pltpu.VMEM((2, ECH), f32),
        pltpu.VMEM((ECH,), f32),
        pltpu.VMEM((3, ECH, H), f32),
        pltpu.VMEM((RPT,), f32),
        pltpu.VMEM_SHARED((N, H), f32),
        pltpu.VMEM_SHARED((N,), f32),
        pltpu.SemaphoreType.DMA,
        pltpu.SemaphoreType.DMA,
        pltpu.SemaphoreType.DMA,
        pltpu.SemaphoreType.DMA,
        pltpu.SemaphoreType.DMA,
        pltpu.SemaphoreType.DMA,
        pltpu.SemaphoreType.DMA,
        pltpu.SemaphoreType.DMA,
    ],
)


def _bf16_bits(x):
    return lax.bitcast_convert_type(
        x.astype(jnp.bfloat16), jnp.uint16).astype(i32)


def _pack_pair(lo, hi):
    return jnp.bitwise_or(lax.shift_left(_bf16_bits(hi), 16),
                          _bf16_bits(lo))


def _head_common(x, wu, wd, au, ad, h2_ref, ab_ref):
    hu = jnp.dot(x, wu, preferred_element_type=f32)
    hd = jnp.dot(x, wd, preferred_element_type=f32)
    h2_ref[0] = hu
    h2_ref[1] = hd
    ab_ref[0, 0, :] = _pack_pair(jnp.sum(hu * au[0:1, :], axis=1),
                                 jnp.sum(hu * au[1:2, :], axis=1))
    ab_ref[0, 1, :] = _pack_pair(jnp.sum(hd * ad[0:1, :], axis=1),
                                 jnp.sum(hd * ad[1:2, :], axis=1))


def _combine(z_ref, den_ref):
    return jax.nn.relu(z_ref[0] / (den_ref[0] + 1e-9)
                       + z_ref[1] / (den_ref[1] + 1e-9))


def _tc_first_body(x_ref, wu_ref, wd_ref, au_ref, ad_ref, h2_ref, ab_ref):
    _head_common(x_ref[...], wu_ref[...], wd_ref[...], au_ref[...],
                 ad_ref[...], h2_ref, ab_ref)


def _tc_mid_body(z_ref, den_ref, wu_ref, wd_ref, au_ref, ad_ref,
                 h2_ref, ab_ref):
    x = _combine(z_ref, den_ref)
    _head_common(x, wu_ref[...], wd_ref[...], au_ref[...], ad_ref[...],
                 h2_ref, ab_ref)


def _tc_tail_body(z_ref, den_ref, wo_ref, bo_ref, o_ref):
    x = _combine(z_ref, den_ref)
    o_ref[...] = jax.nn.sigmoid(
        jnp.dot(x, wo_ref[...], preferred_element_type=f32) + bo_ref[...])


_W_SPEC = pl.BlockSpec((H, H), lambda i: (0, 0))
_A_SPEC = pl.BlockSpec((2, H), lambda i: (0, 0))
_H2_SPEC = pl.BlockSpec((2, BN, H), lambda i: (0, i, 0))
_AB_SPEC = pl.BlockSpec((1, 2, BN), lambda i: (i, 0, 0))
_DEN_SPEC = pl.BlockSpec((2, BN, 1), lambda i: (0, i, 0))
_HEAD_OUT = (
    jax.ShapeDtypeStruct((2, N, H), f32),
    jax.ShapeDtypeStruct((GRID, 2, BN), i32),
)

_tc_first = pl.pallas_call(
    _tc_first_body,
    grid=(GRID,),
    in_specs=[pl.BlockSpec((BN, H), lambda i: (i, 0)),
              _W_SPEC, _W_SPEC, _A_SPEC, _A_SPEC],
    out_specs=(_H2_SPEC, _AB_SPEC),
    out_shape=_HEAD_OUT,
)

_tc_mid = pl.pallas_call(
    _tc_mid_body,
    grid=(GRID,),
    in_specs=[_H2_SPEC, _DEN_SPEC, _W_SPEC, _W_SPEC, _A_SPEC, _A_SPEC],
    out_specs=(_H2_SPEC, _AB_SPEC),
    out_shape=_HEAD_OUT,
)

_tc_tail = pl.pallas_call(
    _tc_tail_body,
    grid=(GRID,),
    in_specs=[_H2_SPEC, _DEN_SPEC,
              pl.BlockSpec((H, C), lambda i: (0, 0)),
              pl.BlockSpec((C,), lambda i: (0,))],
    out_specs=pl.BlockSpec((BN, C), lambda i: (i, 0)),
    out_shape=jax.ShapeDtypeStruct((N, C), f32),
)


def kernel(x_1, up_laplacian_indices, up_laplacian_values,
           down_laplacian_indices, down_laplacian_values,
           Wup, Wdn, Aup, Adn, W_out, b_out):
    idx_u = up_laplacian_indices.astype(i32)
    idx_d = down_laplacian_indices.astype(i32)
    ru, cu = idx_u[0], idx_u[1]
    rd, cd = idx_d[0], idx_d[1]

    h2, ab = _tc_first(x_1, Wup[0], Wdn[0], Aup[0], Adn[0])
    ab = jnp.transpose(ab, (1, 0, 2)).reshape(2 * N)
    z, den = _sc_edge(ru, cu, up_laplacian_values, rd, cd,
                      down_laplacian_values, h2, ab)
    den = den.reshape(2, N, 1)
    h2, ab = _tc_mid(z, den, Wup[1], Wdn[1], Aup[1], Adn[1])
    ab = jnp.transpose(ab, (1, 0, 2)).reshape(2 * N)
    z, den = _sc_edge(ru, cu, up_laplacian_values, rd, cd,
                      down_laplacian_values, h2, ab)
    den = den.reshape(2, N, 1)
    return _tc_tail(z, den, W_out, b_out)


# speedup vs baseline: 1.0038x; 1.0038x over previous
import jax
import jax.numpy as jnp
from jax import lax
from jax.experimental import pallas as pl
from jax.experimental.pallas import tpu as pltpu
from jax.experimental.pallas import tpu_sc as plsc

N = 10000
H = 128
C = 7
NNZ = 320000

NSUB = 16
EPT = NNZ // NSUB
ECH = 80
NCH = EPT // ECH
RPT = 624
TAIL = N - NSUB * RPT

BN = 2000
GRID = N // BN

f32 = jnp.float32
i32 = jnp.int32


def _sc_edge_body(rows_u, cols_u, vals_u, rows_d, cols_d, vals_d, h2, abp,
                  z_out, den_out,
                  ab_t, rows_b, cols_b, vals_b, srows_b, pbuf, coef_b,
                  gath_b, zvec, zacc, den_sh,
                  psem0, psem1, gsem0, gsem1, gsem2, ssem0, ssem1, ssem2):
    core = lax.axis_index("c")
    s = lax.axis_index("s")
    psem = (psem0, psem1)
    gsem = (gsem0, gsem1, gsem2)
    ssem = (ssem0, ssem1, ssem2)

    def _zv(i, _):
        zvec[pl.ds(i * 16, 16)] = jnp.zeros((16,), f32)
        return 0
    lax.fori_loop(0, RPT // 16, _zv, 0)

    def _zg(i, _):
        r = i // 8
        j = i - r * 8
        gath_b[0, r, pl.ds(j * 16, 16)] = jnp.zeros((16,), f32)
        return 0
    lax.fori_loop(0, ECH * 8, _zg, 0)

    pltpu.sync_copy(zvec, den_sh.at[pl.ds(s * RPT, RPT)])

    for k in range(RPT // ECH):
        pltpu.sync_copy(gath_b.at[0],
                        zacc.at[pl.ds(s * RPT + k * ECH, ECH)])
    pltpu.sync_copy(gath_b.at[0, pl.ds(0, RPT % ECH)],
                    zacc.at[pl.ds(s * RPT + RPT // ECH * ECH, RPT % ECH)])

    @pl.when(s == NSUB - 1)
    def _():
        pltpu.sync_copy(zvec.at[pl.ds(0, TAIL)],
                        den_sh.at[pl.ds(NSUB * RPT, TAIL)])
        pltpu.sync_copy(gath_b.at[0, pl.ds(0, TAIL)],
                        zacc.at[pl.ds(NSUB * RPT, TAIL)])

    plsc.subcore_barrier()

    def run_dir(d, rows_h, cols_h, vals_h):
        pltpu.sync_copy(abp.at[pl.ds(d * N, N)], ab_t)

        def pack_start(k, b):
            base = s * EPT + k * ECH
            pltpu.async_copy(rows_h.at[pl.ds(base, ECH)], rows_b.at[b],
                             psem[b])
            pltpu.async_copy(cols_h.at[pl.ds(base, ECH)], cols_b.at[b],
                             psem[b])
            pltpu.async_copy(vals_h.at[pl.ds(base, ECH)], vals_b.at[b],
                             psem[b])

        def pack_wait(b):
            base = s * EPT
            pltpu.make_async_copy(rows_h.at[pl.ds(base, ECH)],
                                  rows_b.at[b], psem[b]).wait()
            pltpu.make_async_copy(cols_h.at[pl.ds(base, ECH)],
                                  cols_b.at[b], psem[b]).wait()
            pltpu.make_async_copy(vals_h.at[pl.ds(base, ECH)],
                                  vals_b.at[b], psem[b]).wait()

        def gather_start(g, b):
            pltpu.async_copy(h2.at[d].at[cols_b.at[b]], gath_b.at[g],
                             gsem[g])

        def gather_wait(g, b):
            pltpu.make_async_copy(h2.at[d].at[cols_b.at[b]], gath_b.at[g],
                                  gsem[g]).wait()

        def scatter_start(g, b):
            pltpu.async_copy(gath_b.at[g], zacc.at[srows_b.at[b]], ssem[g],
                             add=True)
            pltpu.async_copy(pbuf.at[b], den_sh.at[srows_b.at[b]], ssem[g],
                             add=True)

        def scatter_wait(g, b):
            pltpu.make_async_copy(gath_b.at[g], zacc.at[srows_b.at[b]],
                                  ssem[g]).wait()
            pltpu.make_async_copy(pbuf.at[b], den_sh.at[srows_b.at[b]],
                                  ssem[g]).wait()

        def dispatch3(gv, fn):
            for v in range(3):
                @pl.when(gv == v)
                def _():
                    fn(v)

        def compute(b):
            for g in range(ECH // 16):
                sl = pl.ds(g * 16, 16)
                r16 = rows_b[b, sl]
                c16 = cols_b[b, sl]
                srows_b[b, sl] = r16
                tr = plsc.load_gather(ab_t, [r16])
                tc_ = plsc.load_gather(ab_t, [c16])
                av = plsc.bitcast(lax.shift_left(tr, 16), f32)
                bv = plsc.bitcast(
                    jnp.bitwise_and(tc_, jnp.int32(-65536)), f32)
                e = av + bv
                e = jnp.maximum(e, 0.2 * e)
                p = jnp.exp(e)
                pbuf[b, sl] = p
                coef_b[sl] = p * vals_b[b, sl]

        def scale(gv):
            def body(e_i, _):
                g = e_i // 16
                lane = e_i - g * 16
                cvec = coef_b[pl.ds(g * 16, 16)]
                bc = lax.gather(
                    cvec, jnp.full((16, 1), lane, i32),
                    dimension_numbers=lax.GatherDimensionNumbers(
                        offset_dims=(), collapsed_slice_dims=(0,),
                        start_index_map=(0,)),
                    slice_sizes=(1,),
                    mode=lax.GatherScatterMode.PROMISE_IN_BOUNDS)
                for j in range(8):
                    sl2 = pl.ds(j * 16, 16)
                    gath_b[gv, e_i, sl2] = gath_b[gv, e_i, sl2] * bc
                return 0
            lax.fori_loop(0, ECH, body, 0, unroll=4)

        pack_start(0, 0)
        pack_start(1, 1)
        pack_wait(0)
        gather_start(0, 0)

        def loop_body(kk, _):
            for j in range(2):
                b, opp = j, 1 - j
                k = 2 * kk + j
                gv = k % 3

                @pl.when(kk > 0)
                def _():
                    dispatch3((k - 2) % 3,
                              lambda v: scatter_wait(v, b))
                compute(b)
                dispatch3(gv, lambda v: gather_wait(v, b))

                @pl.when(kk < NCH // 2 - 1)
                def _():
                    pack_start(k + 2, b)

                if j == 0:
                    pack_wait(opp)
                    dispatch3((k + 1) % 3,
                              lambda v: gather_start(v, opp))
                else:
                    @pl.when(kk < NCH // 2 - 1)
                    def _():
                        pack_wait(opp)
                        dispatch3((k + 1) % 3,
                                  lambda v: gather_start(v, opp))
                dispatch3(gv, scale)
                dispatch3(gv, lambda v: scatter_start(v, b))
            return 0

        lax.fori_loop(0, NCH // 2, loop_body, 0)
        scatter_wait((NCH - 2) % 3, 0)
        scatter_wait((NCH - 1) % 3, 1)

        plsc.subcore_barrier()
        pltpu.sync_copy(zacc.at[pl.ds(s * RPT, RPT)],
                        z_out.at[d, pl.ds(s * RPT, RPT)])
        pltpu.sync_copy(den_sh.at[pl.ds(s * RPT, RPT)], zvec)
        pltpu.sync_copy(zvec, den_out.at[pl.ds(d * N + s * RPT, RPT)])

        @pl.when(s == NSUB - 1)
        def _():
            pltpu.sync_copy(zacc.at[pl.ds(NSUB * RPT, TAIL)],
                            z_out.at[d, pl.ds(NSUB * RPT, TAIL)])
            pltpu.sync_copy(den_sh.at[pl.ds(NSUB * RPT, TAIL)],
                            zvec.at[pl.ds(0, TAIL)])
            pltpu.sync_copy(zvec.at[pl.ds(0, TAIL)],
                            den_out.at[pl.ds(d * N + NSUB * RPT, TAIL)])

    @pl.when(core == 0)
    def _():
        run_dir(0, rows_u, cols_u, vals_u)

    @pl.when(core == 1)
    def _():
        run_dir(1, rows_d, cols_d, vals_d)


_sc_edge = pl.kernel(
    _sc_edge_body,
    out_type=(
        jax.ShapeDtypeStruct((2, N, H), f32),
        jax.ShapeDtypeStruct((2 * N,), f32),
    ),
    mesh=plsc.VectorSubcoreMesh(core_axis_name="c", subcore_axis_name="s"),
    compiler_params=pltpu.CompilerParams(needs_layout_passes=False),
    scratch_types=[
        pltpu.VMEM((N,), i32),
        pltpu.VMEM((2, ECH), i32),
        pltpu.VMEM((2, ECH), i32),
        pltpu.VMEM((2, ECH), f32),
        pltpu.VMEM((2, ECH), i32),
        pltpu.VMEM((2, ECH), f32),
        pltpu.VMEM((ECH,), f32),
        pltpu.VMEM((3, ECH, H), f32),
        pltpu.VMEM((RPT,), f32),
        pltpu.VMEM_SHARED((N, H), f32),
        pltpu.VMEM_SHARED((N,), f32),
        pltpu.SemaphoreType.DMA,
        pltpu.SemaphoreType.DMA,
        pltpu.SemaphoreType.DMA,
        pltpu.SemaphoreType.DMA,
        pltpu.SemaphoreType.DMA,
        pltpu.SemaphoreType.DMA,
        pltpu.SemaphoreType.DMA,
        pltpu.SemaphoreType.DMA,
    ],
)


def _bf16_bits(x):
    return lax.bitcast_convert_type(
        x.astype(jnp.bfloat16), jnp.uint16).astype(i32)


def _pack_pair(lo, hi):
    return jnp.bitwise_or(lax.shift_left(_bf16_bits(hi), 16),
                          _bf16_bits(lo))


def _head_common(x, wu, wd, au, ad, h2_ref, ab_ref):
    hu = jnp.dot(x, wu, preferred_element_type=f32)
    hd = jnp.dot(x, wd, preferred_element_type=f32)
    h2_ref[0] = hu
    h2_ref[1] = hd
    ab_ref[0, 0, :] = _pack_pair(jnp.sum(hu * au[0:1, :], axis=1),
                                 jnp.sum(hu * au[1:2, :], axis=1))
    ab_ref[0, 1, :] = _pack_pair(jnp.sum(hd * ad[0:1, :], axis=1),
                                 jnp.sum(hd * ad[1:2, :], axis=1))


def _combine(z_ref, den_ref):
    return jax.nn.relu(z_ref[0] / (den_ref[0] + 1e-9)
                       + z_ref[1] / (den_ref[1] + 1e-9))


def _tc_first_body(x_ref, wu_ref, wd_ref, au_ref, ad_ref, h2_ref, ab_ref):
    _head_common(x_ref[...], wu_ref[...], wd_ref[...], au_ref[...],
                 ad_ref[...], h2_ref, ab_ref)


def _tc_mid_body(z_ref, den_ref, wu_ref, wd_ref, au_ref, ad_ref,
                 h2_ref, ab_ref):
    x = _combine(z_ref, den_ref)
    _head_common(x, wu_ref[...], wd_ref[...], au_ref[...], ad_ref[...],
                 h2_ref, ab_ref)


def _tc_tail_body(z_ref, den_ref, wo_ref, bo_ref, o_ref):
    x = _combine(z_ref, den_ref)
    o_ref[...] = jax.nn.sigmoid(
        jnp.dot(x, wo_ref[...], preferred_element_type=f32) + bo_ref[...])


_W_SPEC = pl.BlockSpec((H, H), lambda i: (0, 0))
_A_SPEC = pl.BlockSpec((2, H), lambda i: (0, 0))
_H2_SPEC = pl.BlockSpec((2, BN, H), lambda i: (0, i, 0))
_AB_SPEC = pl.BlockSpec((1, 2, BN), lambda i: (i, 0, 0))
_DEN_SPEC = pl.BlockSpec((2, BN, 1), lambda i: (0, i, 0))
_HEAD_OUT = (
    jax.ShapeDtypeStruct((2, N, H), f32),
    jax.ShapeDtypeStruct((GRID, 2, BN), i32),
)

_tc_first = pl.pallas_call(
    _tc_first_body,
    grid=(GRID,),
    in_specs=[pl.BlockSpec((BN, H), lambda i: (i, 0)),
              _W_SPEC, _W_SPEC, _A_SPEC, _A_SPEC],
    out_specs=(_H2_SPEC, _AB_SPEC),
    out_shape=_HEAD_OUT,
)

_tc_mid = pl.pallas_call(
    _tc_mid_body,
    grid=(GRID,),
    in_specs=[_H2_SPEC, _DEN_SPEC, _W_SPEC, _W_SPEC, _A_SPEC, _A_SPEC],
    out_specs=(_H2_SPEC, _AB_SPEC),
    out_shape=_HEAD_OUT,
)

_tc_tail = pl.pallas_call(
    _tc_tail_body,
    grid=(GRID,),
    in_specs=[_H2_SPEC, _DEN_SPEC,
              pl.BlockSpec((H, C), lambda i: (0, 0)),
              pl.BlockSpec((C,), lambda i: (0,))],
    out_specs=pl.BlockSpec((BN, C), lambda i: (i, 0)),
    out_shape=jax.ShapeDtypeStruct((N, C), f32),
)


def kernel(x_1, up_laplacian_indices, up_laplacian_values,
           down_laplacian_indices, down_laplacian_values,
           Wup, Wdn, Aup, Adn, W_out, b_out):
    idx_u = up_laplacian_indices.astype(i32)
    idx_d = down_laplacian_indices.astype(i32)
    ru, cu = idx_u[0], idx_u[1]
    rd, cd = idx_d[0], idx_d[1]

    h2, ab = _tc_first(x_1, Wup[0], Wdn[0], Aup[0], Adn[0])
    ab = jnp.transpose(ab, (1, 0, 2)).reshape(2 * N)
    z, den = _sc_edge(ru, cu, up_laplacian_values, rd, cd,
                      down_laplacian_values, h2, ab)
    den = den.reshape(2, N, 1)
    h2, ab = _tc_mid(z, den, Wup[1], Wdn[1], Aup[1], Adn[1])
    ab = jnp.transpose(ab, (1, 0, 2)).reshape(2 * N)
    z, den = _sc_edge(ru, cu, up_laplacian_values, rd, cd,
                      down_laplacian_values, h2, ab)
    den = den.reshape(2, N, 1)
    return _tc_tail(z, den, W_out, b_out)
